# baseline (speedup 1.0000x reference)
import functools

import jax
import jax.numpy as jnp
from jax import lax
from jax.experimental import pallas as pl
from jax.experimental.pallas import tpu as pltpu
from jax.experimental.pallas import tpu_sc as plsc

D = 64
BATCH, SEQ = 4096, 50
NC, NS = 2, 16
NW = NC * NS
BW = BATCH // NW
ET, ES = D // 8, 8
BT = BATCH // 128
NB = 3


def _build():
  mesh = plsc.VectorSubcoreMesh(core_axis_name="c", subcore_axis_name="s")

  @functools.partial(
      pl.kernel,
      mesh=mesh,
      compiler_params=pltpu.CompilerParams(
          use_tc_tiling_on_sc=True, needs_layout_passes=False),
      out_type=jax.ShapeDtypeStruct((SEQ, ET, BT, ES, BW), jnp.float32),
      scratch_types=[
          pltpu.VMEM((SEQ, BW), jnp.int32),
          pltpu.VMEM((SEQ, BW), jnp.int32),
          pltpu.VMEM((SEQ, BW), jnp.int32),
          pltpu.VMEM((NB, BW, 2 * D), jnp.float32),
          pltpu.VMEM((NB, D, BW), jnp.float32),
          pltpu.SemaphoreType.DMA,
          pltpu.SemaphoreType.DMA,
          pltpu.SemaphoreType.DMA,
          pltpu.SemaphoreType.DMA,
          pltpu.SemaphoreType.DMA,
          pltpu.SemaphoreType.DMA,
      ],
  )
  def emb(idx_hbm, table_hbm, out_hbm, idx_v, pair_v, half_v, rows_v,
          tblk_v, gsem0, gsem1, gsem2, osem0, osem1, osem2):
    wid = lax.axis_index("s") * NC + lax.axis_index("c")
    col = pl.multiple_of(wid * BW, BW)
    pltpu.sync_copy(idx_hbm.at[:, pl.ds(col, BW)], idx_v)

    gsems = (gsem0, gsem1, gsem2)
    osems = (osem0, osem1, osem2)
    lane = lax.iota(jnp.int32, 16)
    qs = [(lane + j) & 15 for j in range(16)]

    def split_body(s, carry):
      for g in range(BW // 16):
        v = idx_v[s, pl.ds(16 * g, 16)]
        pair_v[s, pl.ds(16 * g, 16)] = v >> 1
        half_v[s, pl.ds(16 * g, 16)] = (v & 1) * D
      return carry

    lax.fori_loop(0, SEQ, split_body, 0)

    def start_gather(s, b):
      pltpu.make_async_copy(
          table_hbm.at[pair_v.at[s]], rows_v.at[b], gsems[b]).start()

    def wait_gather(s, b):
      pltpu.make_async_copy(
          table_hbm.at[pair_v.at[s]], rows_v.at[b], gsems[b]).wait()

    def transpose(s, b):
      rows = rows_v.at[b]
      tblk = tblk_v.at[b]

      def g_body(g, carry):
        row_idx = lane + g * 16
        half = half_v[s, pl.ds(g * 16, 16)]
        for c in range(D // 16):
          vs = [
              plsc.load_gather(rows, [row_idx, half + (qs[j] + 16 * c)])
              for j in range(16)
          ]
          for j in range(16):
            plsc.store_scatter(
                tblk, [qs[j] + 16 * c, row_idx], vs[j])
        return carry

      lax.fori_loop(0, BW // 16, g_body, 0)

    def start_out(s, b):
      for tr in range(ET):
        pltpu.make_async_copy(
            tblk_v.at[b, pl.ds(tr * ES, ES), :],
            out_hbm.at[s, tr, wid],
            osems[b]).start()

    def wait_out(s, b):
      for tr in range(ET):
        pltpu.make_async_copy(
            tblk_v.at[b, pl.ds(tr * ES, ES), :],
            out_hbm.at[s, tr, wid],
            osems[b]).wait()

    start_gather(0, 0)
    start_gather(1, 1)
    start_gather(2, 2)

    STEADY = (SEQ // NB) - 1

    def body(i, carry):
      for b in range(NB):
        s = NB * i + b
        wait_gather(s, b)

        @pl.when(i >= 1)
        def _():
          wait_out(s, b)

        transpose(s, b)
        start_gather(s + NB, b)
        start_out(s, b)
      return carry

    lax.fori_loop(0, STEADY, body, 0)
    for s in range(NB * STEADY, SEQ):
      b = s % NB
      wait_gather(s, b)
      wait_out(s, b)
      transpose(s, b)
      if s + NB < SEQ:
        start_gather(s + NB, b)
      start_out(s, b)
    for s in range(SEQ - NB, SEQ):
      wait_out(s, s % NB)

  return emb


_emb = _build()


def kernel(item_inputs, table):
  idx_t = item_inputs.T.astype(jnp.int32)
  table2 = table.reshape(50000, 2 * D)
  out5 = _emb(idx_t, table2)
  out = out5.transpose(2, 4, 0, 1, 3).reshape(BATCH, SEQ, D)
  return out
